# Initial kernel scaffold; baseline (speedup 1.0000x reference)
import jax
import jax.numpy as jnp
from jax.experimental import pallas as pl


def kernel(x, edge_index, edge_weight, W1, b1, W2, b2):
    raise NotImplementedError("write your pallas kernel here")


# speedup vs baseline: 14.3092x; 14.3092x over previous
import functools

import jax
import jax.numpy as jnp
from jax import lax
from jax.experimental import pallas as pl
from jax.experimental.pallas import tpu as pltpu
from jax.experimental.pallas import tpu_sc as plsc

N = 10000
E = 320000
D = 128

NC = 2
NS = 16
NW = NC * NS
EPW = E // NW
CHUNK = 80
NCHUNK = EPW // CHUNK
TROWS = 624
ZROWS = 16
BN = 1000


def _mesh():
    return plsc.VectorSubcoreMesh(core_axis_name="c", subcore_axis_name="s")


@functools.partial(
    pl.kernel,
    mesh=_mesh(),
    out_type=jax.ShapeDtypeStruct((NW, N), jnp.float32),
    compiler_params=pltpu.CompilerParams(needs_layout_passes=False),
    scratch_types=[
        pltpu.VMEM((EPW,), jnp.int32),
        pltpu.VMEM((EPW,), jnp.float32),
        pltpu.VMEM((N,), jnp.float32),
    ],
)
def _deg_kernel(dst_hbm, ew_hbm, deg_out, dstv, eww, degv):
    cid = lax.axis_index("c")
    sid = lax.axis_index("s")
    wid = cid * NS + sid
    base = wid * EPW

    z16 = jnp.zeros((16,), jnp.float32)

    def zero_body(i, carry):
        degv[pl.ds(i * 16, 16)] = z16
        return carry

    lax.fori_loop(0, N // 16, zero_body, 0)

    pltpu.sync_copy(dst_hbm.at[pl.ds(base, EPW)], dstv)
    pltpu.sync_copy(ew_hbm.at[pl.ds(base, EPW)], eww)

    def grp_body(g, carry):
        d16 = dstv[pl.ds(g * 16, 16)]
        w16 = eww[pl.ds(g * 16, 16)]
        plsc.addupdate_scatter(degv, [d16], w16)
        return carry

    lax.fori_loop(0, EPW // 16, grp_body, 0)

    pltpu.sync_copy(degv, deg_out.at[wid])


@functools.partial(
    pl.kernel,
    mesh=_mesh(),
    out_type=jax.ShapeDtypeStruct((NC, N, D), jnp.float32),
    compiler_params=pltpu.CompilerParams(needs_layout_passes=False),
    scratch_types=[
        pltpu.VMEM((EPW,), jnp.int32),
        pltpu.VMEM((NCHUNK, CHUNK), jnp.int32),
        pltpu.VMEM((EPW,), jnp.float32),
        pltpu.VMEM((CHUNK, D), jnp.float32),
        pltpu.VMEM((ZROWS, D), jnp.float32),
        pltpu.VMEM_SHARED((N, D), jnp.float32),
        pltpu.SemaphoreType.DMA,
    ],
)
def _agg_kernel(h_hbm, src_hbm, dst_hbm, ew_hbm, agg_out,
                srcv, dstv, eww, rows, zbuf, agg_s, sem):
    cid = lax.axis_index("c")
    sid = lax.axis_index("s")
    wid = cid * NS + sid
    base = wid * EPW

    z16 = jnp.zeros((16,), jnp.float32)

    def zzero(i, carry):
        for j in range(D // 16):
            zbuf[i, pl.ds(j * 16, 16)] = z16
        return carry

    lax.fori_loop(0, ZROWS, zzero, 0)

    def szero(k, carry):
        pltpu.sync_copy(zbuf, agg_s.at[pl.ds(sid * TROWS + k * ZROWS, ZROWS)])
        return carry

    lax.fori_loop(0, TROWS // ZROWS, szero, 0)

    @pl.when(sid == NS - 1)
    def _():
        pltpu.sync_copy(zbuf, agg_s.at[pl.ds(NS * TROWS, ZROWS)])

    pltpu.sync_copy(src_hbm.at[pl.ds(base, EPW)], srcv)
    pltpu.sync_copy(ew_hbm.at[pl.ds(base, EPW)], eww)

    def dstage(c, carry):
        pltpu.sync_copy(dst_hbm.at[pl.ds(base + c * CHUNK, CHUNK)], dstv.at[c])
        return carry

    lax.fori_loop(0, NCHUNK, dstage, 0)

    plsc.subcore_barrier()

    def chunk_body(c, carry):
        pltpu.async_copy(h_hbm.at[srcv.at[pl.ds(c * CHUNK, CHUNK)]],
                         rows, sem).wait()

        def grp_body(g, carry2):
            ew16 = eww[pl.ds(c * CHUNK + g * 16, 16)]
            for r in range(16):
                w = ew16.at[jnp.full((16,), r, jnp.int32)].get(
                    mode="promise_in_bounds")
                row = g * 16 + r
                for j in range(D // 16):
                    sl = pl.ds(j * 16, 16)
                    rows[row, sl] = rows[row, sl] * w
            return carry2

        lax.fori_loop(0, CHUNK // 16, grp_body, 0)

        pltpu.sync_copy(rows, agg_s.at[dstv.at[c]], add=True)
        return carry

    lax.fori_loop(0, NCHUNK, chunk_body, 0)

    plsc.subcore_barrier()

    pltpu.sync_copy(agg_s.at[pl.ds(sid * TROWS, TROWS)],
                    agg_out.at[cid, pl.ds(sid * TROWS, TROWS)])

    @pl.when(sid == NS - 1)
    def _():
        pltpu.sync_copy(agg_s.at[pl.ds(NS * TROWS, ZROWS)],
                        agg_out.at[cid, pl.ds(NS * TROWS, ZROWS)])


def _dis_from_partials(pt_blk):
    deg = 1.0 + jnp.sum(pt_blk, axis=1, keepdims=True)
    return lax.rsqrt(deg)


def _hprime_body(x_ref, w_ref, pt_ref, o_ref):
    dis = _dis_from_partials(pt_ref[...])
    h = jnp.dot(x_ref[...], w_ref[...], preferred_element_type=jnp.float32)
    o_ref[...] = h * dis


def _mid_body(agg_ref, hp_ref, pt_ref, b_ref, w_ref, o_ref):
    dis = _dis_from_partials(pt_ref[...])
    tot = agg_ref[0] + agg_ref[1] + hp_ref[...]
    h1 = jnp.maximum(tot * dis + b_ref[...], 0.0)
    o_ref[...] = jnp.dot(h1, w_ref[...], preferred_element_type=jnp.float32) * dis


def _final_body(agg_ref, hp_ref, pt_ref, b_ref, o_ref):
    dis = _dis_from_partials(pt_ref[...])
    tot = agg_ref[0] + agg_ref[1] + hp_ref[...]
    o_ref[...] = tot * dis + b_ref[...]


_row_spec = pl.BlockSpec((BN, D), lambda i: (i, 0))
_pt_spec = pl.BlockSpec((BN, NW), lambda i: (i, 0))
_w_spec = pl.BlockSpec((D, D), lambda i: (0, 0))
_b_spec = pl.BlockSpec((1, D), lambda i: (0, 0))
_agg_spec = pl.BlockSpec((NC, BN, D), lambda i: (0, i, 0))
_out_struct = jax.ShapeDtypeStruct((N, D), jnp.float32)
_GRID = (N // BN,)

_hprime_call = pl.pallas_call(
    _hprime_body, grid=_GRID,
    in_specs=[_row_spec, _w_spec, _pt_spec],
    out_specs=_row_spec, out_shape=_out_struct)

_mid_call = pl.pallas_call(
    _mid_body, grid=_GRID,
    in_specs=[_agg_spec, _row_spec, _pt_spec, _b_spec, _w_spec],
    out_specs=_row_spec, out_shape=_out_struct)

_final_call = pl.pallas_call(
    _final_body, grid=_GRID,
    in_specs=[_agg_spec, _row_spec, _pt_spec, _b_spec],
    out_specs=_row_spec, out_shape=_out_struct)


def kernel(x, edge_index, edge_weight, W1, b1, W2, b2):
    src = edge_index[0]
    dst = edge_index[1]
    b1r = b1.reshape(1, D)
    b2r = b2.reshape(1, D)

    deg_partials = _deg_kernel(dst, edge_weight)
    pt = deg_partials.T

    h1p = _hprime_call(x, W1, pt)
    agg1 = _agg_kernel(h1p, src, dst, edge_weight)
    h2p = _mid_call(agg1, h1p, pt, b1r, W2)
    agg2 = _agg_kernel(h2p, src, dst, edge_weight)
    return _final_call(agg2, h2p, pt, b2r)


# speedup vs baseline: 23.6645x; 1.6538x over previous
import functools

import jax
import jax.numpy as jnp
from jax import lax
from jax.experimental import pallas as pl
from jax.experimental.pallas import tpu as pltpu
from jax.experimental.pallas import tpu_sc as plsc

N = 10000
E = 320000
D = 128

NC = 2
NS = 16
NW = NC * NS
EPW = E // NW
CHUNK = 80
PAIR = 2 * CHUNK
NPAIR = EPW // PAIR
TAILC = 2 * NPAIR
TROWS = 624
ZROWS = 16
BN = 1000


def _mesh():
    return plsc.VectorSubcoreMesh(core_axis_name="c", subcore_axis_name="s")


@functools.partial(
    pl.kernel,
    mesh=_mesh(),
    out_type=jax.ShapeDtypeStruct((NW, N), jnp.float32),
    compiler_params=pltpu.CompilerParams(needs_layout_passes=False),
    scratch_types=[
        pltpu.VMEM((EPW,), jnp.int32),
        pltpu.VMEM((EPW,), jnp.float32),
        pltpu.VMEM((N,), jnp.float32),
    ],
)
def _deg_kernel(dst_hbm, ew_hbm, deg_out, dstv, eww, degv):
    cid = lax.axis_index("c")
    sid = lax.axis_index("s")
    wid = cid * NS + sid
    base = wid * EPW

    z16 = jnp.zeros((16,), jnp.float32)

    def zero_body(i, carry):
        degv[pl.ds(i * 16, 16)] = z16
        return carry

    lax.fori_loop(0, N // 16, zero_body, 0)

    pltpu.sync_copy(dst_hbm.at[pl.ds(base, EPW)], dstv)
    pltpu.sync_copy(ew_hbm.at[pl.ds(base, EPW)], eww)

    def grp_body(g, carry):
        d16 = dstv[pl.ds(g * 16, 16)]
        w16 = eww[pl.ds(g * 16, 16)]
        plsc.addupdate_scatter(degv, [d16], w16)
        return carry

    lax.fori_loop(0, EPW // 16, grp_body, 0)

    pltpu.sync_copy(degv, deg_out.at[wid])


@functools.partial(
    pl.kernel,
    mesh=_mesh(),
    out_type=jax.ShapeDtypeStruct((NC, N, D), jnp.float32),
    compiler_params=pltpu.CompilerParams(needs_layout_passes=False),
    scratch_types=[
        pltpu.VMEM((PAIR,), jnp.int32),
        pltpu.VMEM((PAIR,), jnp.int32),
        pltpu.VMEM((2 * PAIR, D), jnp.float32),
        pltpu.VMEM((CHUNK,), jnp.int32),
        pltpu.VMEM((CHUNK,), jnp.int32),
        pltpu.VMEM((CHUNK,), jnp.int32),
        pltpu.VMEM((CHUNK,), jnp.int32),
        pltpu.VMEM((2 * PAIR,), jnp.float32),
        pltpu.VMEM_SHARED((N, D), jnp.float32),
        pltpu.SemaphoreType.DMA,
        pltpu.SemaphoreType.DMA,
        pltpu.SemaphoreType.DMA,
        pltpu.SemaphoreType.DMA,
        pltpu.SemaphoreType.DMA,
        pltpu.SemaphoreType.DMA,
        pltpu.SemaphoreType.DMA,
        pltpu.SemaphoreType.DMA,
        pltpu.SemaphoreType.DMA,
    ],
)
def _agg_kernel(h_hbm, src_hbm, dst_hbm, ew_hbm, zeros_hbm, agg_out,
                sp0, sp1, rowsv, d0, d1, d2, d3, ewp, agg_s,
                g0, g1, s0, s1, e0, e1, r0, r1, zsem):
    cid = lax.axis_index("c")
    sid = lax.axis_index("s")
    wid = cid * NS + sid
    base = wid * EPW
    gsem = (g0, g1)
    ssem = (s0, s1)
    esem = (e0, e1)
    rsem = (r0, r1)
    srcp = (sp0, sp1)
    dstp = (d0, d1, d2, d3)

    pltpu.async_copy(zeros_hbm.at[pl.ds(sid * TROWS, TROWS)],
                     agg_s.at[pl.ds(sid * TROWS, TROWS)], zsem)

    @pl.when(sid == NS - 1)
    def _():
        pltpu.async_copy(zeros_hbm.at[pl.ds(NS * TROWS, ZROWS)],
                         agg_s.at[pl.ds(NS * TROWS, ZROWS)], zsem)

    def _rows(b, q):
        return rowsv.at[pl.ds(b * PAIR + q * CHUNK, CHUNK)]

    def _src_start(k, b):
        pltpu.async_copy(src_hbm.at[pl.ds(base + k * PAIR, PAIR)],
                         srcp[b], rsem[b])

    def _src_wait(k, b):
        pltpu.make_async_copy(src_hbm.at[pl.ds(base + k * PAIR, PAIR)],
                              srcp[b], rsem[b]).wait()

    def _gather_start(k, b):
        for q in range(2):
            pltpu.async_copy(h_hbm.at[srcp[b].at[pl.ds(q * CHUNK, CHUNK)]],
                             _rows(b, q), gsem[b])

    def _gather_wait(k, b):
        for q in range(2):
            pltpu.make_async_copy(
                h_hbm.at[srcp[b].at[pl.ds(q * CHUNK, CHUNK)]],
                _rows(b, q), gsem[b]).wait()

    def _dstew_start(k, b):
        for q in range(2):
            pltpu.async_copy(
                dst_hbm.at[pl.ds(base + (2 * k + q) * CHUNK, CHUNK)],
                dstp[2 * b + q], esem[b])
        pltpu.async_copy(ew_hbm.at[pl.ds(base + k * PAIR, PAIR)],
                         ewp.at[pl.ds(b * PAIR, PAIR)], esem[b])

    def _dstew_wait(k, b):
        for q in range(2):
            pltpu.make_async_copy(
                dst_hbm.at[pl.ds(base + (2 * k + q) * CHUNK, CHUNK)],
                dstp[2 * b + q], esem[b]).wait()
        pltpu.make_async_copy(ew_hbm.at[pl.ds(base + k * PAIR, PAIR)],
                              ewp.at[pl.ds(b * PAIR, PAIR)], esem[b]).wait()

    def _scatter_start(k, b):
        for q in range(2):
            pltpu.async_copy(_rows(b, q), agg_s.at[dstp[2 * b + q]],
                             ssem[b], add=True)

    def _scatter_wait(k, b):
        for q in range(2):
            pltpu.make_async_copy(_rows(b, q), agg_s.at[dstp[2 * b + q]],
                                  ssem[b]).wait()

    def _scale_groups(rbase, ngroups):
        def grp_body(g, carry2):
            ew16 = ewp[pl.ds(rbase + g * 16, 16)]
            for r in range(16):
                w = ew16.at[jnp.full((16,), r, jnp.int32)].get(
                    mode="promise_in_bounds")
                row = rbase + g * 16 + r
                for j in range(D // 16):
                    sl = pl.ds(j * 16, 16)
                    rowsv[row, sl] = rowsv[row, sl] * w
            return carry2

        lax.fori_loop(0, ngroups, grp_body, 0)

    _src_start(0, 0)
    _src_wait(0, 0)
    _gather_start(0, 0)
    _dstew_start(0, 0)
    _src_start(1, 1)

    pltpu.make_async_copy(zeros_hbm.at[pl.ds(sid * TROWS, TROWS)],
                          agg_s.at[pl.ds(sid * TROWS, TROWS)], zsem).wait()

    @pl.when(sid == NS - 1)
    def _():
        pltpu.make_async_copy(zeros_hbm.at[pl.ds(NS * TROWS, ZROWS)],
                              agg_s.at[pl.ds(NS * TROWS, ZROWS)], zsem).wait()

    plsc.subcore_barrier()

    def pair_body(k, carry):
        par = lax.rem(k, 2)
        for b in range(2):
            @pl.when(par == b)
            def _():
                _gather_wait(k, b)
                _dstew_wait(k, b)

        _scale_groups(par * PAIR, PAIR // 16)

        for b in range(2):
            @pl.when(par == b)
            def _():
                _scatter_start(k, b)
                nb = 1 - b

                @pl.when(k + 1 < NPAIR)
                def _():
                    @pl.when(k >= 1)
                    def _():
                        _scatter_wait(k - 1, nb)

                    _src_wait(k + 1, nb)
                    _gather_start(k + 1, nb)
                    _dstew_start(k + 1, nb)

                    @pl.when(k + 2 < NPAIR)
                    def _():
                        _src_start(k + 2, b)
        return carry

    lax.fori_loop(0, NPAIR, pair_body, 0)

    _scatter_wait(NPAIR - 2, (NPAIR - 2) % 2)
    _scatter_wait(NPAIR - 1, (NPAIR - 1) % 2)

    tbase = base + TAILC * CHUNK
    pltpu.async_copy(src_hbm.at[pl.ds(tbase, CHUNK)],
                     sp0.at[pl.ds(0, CHUNK)], rsem[0])
    pltpu.async_copy(dst_hbm.at[pl.ds(tbase, CHUNK)], d0, esem[0])
    pltpu.async_copy(ew_hbm.at[pl.ds(tbase, CHUNK)],
                     ewp.at[pl.ds(0, CHUNK)], esem[0])
    pltpu.make_async_copy(src_hbm.at[pl.ds(tbase, CHUNK)],
                          sp0.at[pl.ds(0, CHUNK)], rsem[0]).wait()
    pltpu.make_async_copy(dst_hbm.at[pl.ds(tbase, CHUNK)], d0,
                          esem[0]).wait()
    pltpu.make_async_copy(ew_hbm.at[pl.ds(tbase, CHUNK)],
                          ewp.at[pl.ds(0, CHUNK)], esem[0]).wait()
    pltpu.async_copy(h_hbm.at[sp0.at[pl.ds(0, CHUNK)]],
                     _rows(0, 0), gsem[0])
    pltpu.make_async_copy(h_hbm.at[sp0.at[pl.ds(0, CHUNK)]],
                          _rows(0, 0), gsem[0]).wait()
    _scale_groups(0, CHUNK // 16)
    pltpu.async_copy(_rows(0, 0), agg_s.at[d0], ssem[0], add=True)
    pltpu.make_async_copy(_rows(0, 0), agg_s.at[d0], ssem[0]).wait()

    plsc.subcore_barrier()

    pltpu.sync_copy(agg_s.at[pl.ds(sid * TROWS, TROWS)],
                    agg_out.at[cid, pl.ds(sid * TROWS, TROWS)])

    @pl.when(sid == NS - 1)
    def _():
        pltpu.sync_copy(agg_s.at[pl.ds(NS * TROWS, ZROWS)],
                        agg_out.at[cid, pl.ds(NS * TROWS, ZROWS)])


def _dis_from_partials(pt_blk):
    deg = 1.0 + jnp.sum(pt_blk, axis=1, keepdims=True)
    return lax.rsqrt(deg)


def _hprime_body(x_ref, w_ref, pt_ref, o_ref):
    dis = _dis_from_partials(pt_ref[...])
    h = jnp.dot(x_ref[...], w_ref[...], preferred_element_type=jnp.float32)
    o_ref[...] = h * dis


def _mid_body(agg_ref, hp_ref, pt_ref, b_ref, w_ref, o_ref):
    dis = _dis_from_partials(pt_ref[...])
    tot = agg_ref[0] + agg_ref[1] + hp_ref[...]
    h1 = jnp.maximum(tot * dis + b_ref[...], 0.0)
    o_ref[...] = jnp.dot(h1, w_ref[...], preferred_element_type=jnp.float32) * dis


def _final_body(agg_ref, hp_ref, pt_ref, b_ref, o_ref):
    dis = _dis_from_partials(pt_ref[...])
    tot = agg_ref[0] + agg_ref[1] + hp_ref[...]
    o_ref[...] = tot * dis + b_ref[...]


_row_spec = pl.BlockSpec((BN, D), lambda i: (i, 0))
_pt_spec = pl.BlockSpec((BN, NW), lambda i: (i, 0))
_w_spec = pl.BlockSpec((D, D), lambda i: (0, 0))
_b_spec = pl.BlockSpec((1, D), lambda i: (0, 0))
_agg_spec = pl.BlockSpec((NC, BN, D), lambda i: (0, i, 0))
_out_struct = jax.ShapeDtypeStruct((N, D), jnp.float32)
_GRID = (N // BN,)

_hprime_call = pl.pallas_call(
    _hprime_body, grid=_GRID,
    in_specs=[_row_spec, _w_spec, _pt_spec],
    out_specs=_row_spec, out_shape=_out_struct)

_mid_call = pl.pallas_call(
    _mid_body, grid=_GRID,
    in_specs=[_agg_spec, _row_spec, _pt_spec, _b_spec, _w_spec],
    out_specs=_row_spec, out_shape=_out_struct)

_final_call = pl.pallas_call(
    _final_body, grid=_GRID,
    in_specs=[_agg_spec, _row_spec, _pt_spec, _b_spec],
    out_specs=_row_spec, out_shape=_out_struct)


def kernel(x, edge_index, edge_weight, W1, b1, W2, b2):
    src = edge_index[0]
    dst = edge_index[1]
    b1r = b1.reshape(1, D)
    b2r = b2.reshape(1, D)
    zeros = jnp.zeros((N, D), jnp.float32)

    deg_partials = _deg_kernel(dst, edge_weight)
    pt = deg_partials.T

    h1p = _hprime_call(x, W1, pt)
    agg1 = _agg_kernel(h1p, src, dst, edge_weight, zeros)
    h2p = _mid_call(agg1, h1p, pt, b1r, W2)
    agg2 = _agg_kernel(h2p, src, dst, edge_weight, zeros)
    return _final_call(agg2, h2p, pt, b2r)


# speedup vs baseline: 30.3679x; 1.2833x over previous
import functools

import jax
import jax.numpy as jnp
from jax import lax
from jax.experimental import pallas as pl
from jax.experimental.pallas import tpu as pltpu
from jax.experimental.pallas import tpu_sc as plsc

N = 10000
E = 320000
D = 128

NC = 2
NS = 16
NW = NC * NS
EPW = E // NW
CHUNK = 80
NCHUNK = EPW // CHUNK
TROWS = 624
ZROWS = 16
BN = 1000


def _mesh():
    return plsc.VectorSubcoreMesh(core_axis_name="c", subcore_axis_name="s")


@functools.partial(
    pl.kernel,
    mesh=_mesh(),
    out_type=jax.ShapeDtypeStruct((NW, N), jnp.float32),
    compiler_params=pltpu.CompilerParams(needs_layout_passes=False),
    scratch_types=[
        pltpu.VMEM((EPW,), jnp.int32),
        pltpu.VMEM((EPW,), jnp.float32),
        pltpu.VMEM((N,), jnp.float32),
    ],
)
def _deg_kernel(dst_hbm, ew_hbm, deg_out, dstv, eww, degv):
    cid = lax.axis_index("c")
    sid = lax.axis_index("s")
    wid = cid * NS + sid
    base = wid * EPW

    z16 = jnp.zeros((16,), jnp.float32)

    def zero_body(i, carry):
        degv[pl.ds(i * 16, 16)] = z16
        return carry

    lax.fori_loop(0, N // 16, zero_body, 0)

    pltpu.sync_copy(dst_hbm.at[pl.ds(base, EPW)], dstv)
    pltpu.sync_copy(ew_hbm.at[pl.ds(base, EPW)], eww)

    def grp_body(g, carry):
        d16 = dstv[pl.ds(g * 16, 16)]
        w16 = eww[pl.ds(g * 16, 16)]
        plsc.addupdate_scatter(degv, [d16], w16)
        return carry

    lax.fori_loop(0, EPW // 16, grp_body, 0)

    pltpu.sync_copy(degv, deg_out.at[wid])


@functools.partial(
    pl.kernel,
    mesh=_mesh(),
    out_type=jax.ShapeDtypeStruct((NC, N, D), jnp.float32),
    compiler_params=pltpu.CompilerParams(needs_layout_passes=False),
    scratch_types=[
        pltpu.VMEM((CHUNK,), jnp.int32),
        pltpu.VMEM((CHUNK,), jnp.int32),
        pltpu.VMEM((CHUNK,), jnp.int32),
        pltpu.VMEM((CHUNK,), jnp.int32),
        pltpu.VMEM((4 * CHUNK, D), jnp.float32),
        pltpu.VMEM((CHUNK,), jnp.int32),
        pltpu.VMEM((CHUNK,), jnp.int32),
        pltpu.VMEM((CHUNK,), jnp.int32),
        pltpu.VMEM((CHUNK,), jnp.int32),
        pltpu.VMEM((4 * CHUNK,), jnp.float32),
        pltpu.VMEM_SHARED((N, D), jnp.float32),
        pltpu.SemaphoreType.DMA,
        pltpu.SemaphoreType.DMA,
        pltpu.SemaphoreType.DMA,
        pltpu.SemaphoreType.DMA,
        pltpu.SemaphoreType.DMA,
        pltpu.SemaphoreType.DMA,
        pltpu.SemaphoreType.DMA,
        pltpu.SemaphoreType.DMA,
        pltpu.SemaphoreType.DMA,
        pltpu.SemaphoreType.DMA,
        pltpu.SemaphoreType.DMA,
        pltpu.SemaphoreType.DMA,
        pltpu.SemaphoreType.DMA,
        pltpu.SemaphoreType.DMA,
        pltpu.SemaphoreType.DMA,
        pltpu.SemaphoreType.DMA,
        pltpu.SemaphoreType.DMA,
    ],
)
def _agg_kernel(h_hbm, src_hbm, dst_hbm, ew_hbm, zeros_hbm, agg_out,
                sp0, sp1, sp2, sp3, rowsv, d0, d1, d2, d3, eww, agg_s,
                g0, g1, g2, g3, s0, s1, s2, s3, e0, e1, e2, e3,
                r0, r1, r2, r3, zsem):
    cid = lax.axis_index("c")
    sid = lax.axis_index("s")
    wid = cid * NS + sid
    base = wid * EPW
    srcb = (sp0, sp1, sp2, sp3)
    dstb = (d0, d1, d2, d3)
    gsem = (g0, g1, g2, g3)
    ssem = (s0, s1, s2, s3)
    esem = (e0, e1, e2, e3)
    rsem = (r0, r1, r2, r3)

    pltpu.async_copy(zeros_hbm.at[pl.ds(sid * TROWS, TROWS)],
                     agg_s.at[pl.ds(sid * TROWS, TROWS)], zsem)

    @pl.when(sid == NS - 1)
    def _():
        pltpu.async_copy(zeros_hbm.at[pl.ds(NS * TROWS, ZROWS)],
                         agg_s.at[pl.ds(NS * TROWS, ZROWS)], zsem)

    def _rows(b):
        return rowsv.at[pl.ds(b * CHUNK, CHUNK)]

    def _src_start(c, b):
        pltpu.async_copy(src_hbm.at[pl.ds(base + c * CHUNK, CHUNK)],
                         srcb[b], rsem[b])

    def _src_wait(c, b):
        pltpu.make_async_copy(src_hbm.at[pl.ds(base + c * CHUNK, CHUNK)],
                              srcb[b], rsem[b]).wait()

    def _gather_start(c, b):
        pltpu.async_copy(h_hbm.at[srcb[b]], _rows(b), gsem[b])

    def _gather_wait(c, b):
        pltpu.make_async_copy(h_hbm.at[srcb[b]], _rows(b), gsem[b]).wait()

    def _dstew_start(c, b):
        pltpu.async_copy(dst_hbm.at[pl.ds(base + c * CHUNK, CHUNK)],
                         dstb[b], esem[b])
        pltpu.async_copy(ew_hbm.at[pl.ds(base + c * CHUNK, CHUNK)],
                         eww.at[pl.ds(b * CHUNK, CHUNK)], esem[b])

    def _dstew_wait(c, b):
        pltpu.make_async_copy(dst_hbm.at[pl.ds(base + c * CHUNK, CHUNK)],
                              dstb[b], esem[b]).wait()
        pltpu.make_async_copy(ew_hbm.at[pl.ds(base + c * CHUNK, CHUNK)],
                              eww.at[pl.ds(b * CHUNK, CHUNK)], esem[b]).wait()

    def _scatter_start(c, b):
        pltpu.async_copy(_rows(b), agg_s.at[dstb[b]], ssem[b], add=True)

    def _scatter_wait(c, b):
        pltpu.make_async_copy(_rows(b), agg_s.at[dstb[b]], ssem[b]).wait()

    _src_start(0, 0)
    _src_start(1, 1)
    _src_wait(0, 0)
    _gather_start(0, 0)
    _dstew_start(0, 0)
    _src_wait(1, 1)
    _gather_start(1, 1)
    _dstew_start(1, 1)
    _src_start(2, 2)
    _src_start(3, 3)

    pltpu.make_async_copy(zeros_hbm.at[pl.ds(sid * TROWS, TROWS)],
                          agg_s.at[pl.ds(sid * TROWS, TROWS)], zsem).wait()

    @pl.when(sid == NS - 1)
    def _():
        pltpu.make_async_copy(zeros_hbm.at[pl.ds(NS * TROWS, ZROWS)],
                              agg_s.at[pl.ds(NS * TROWS, ZROWS)], zsem).wait()

    plsc.subcore_barrier()

    def chunk_body(c, carry):
        par = lax.rem(c, 4)
        for b in range(4):
            @pl.when(par == b)
            def _():
                _gather_wait(c, b)
                _dstew_wait(c, b)

        rbase = par * CHUNK

        def grp_body(g, carry2):
            ew16 = eww[pl.ds(rbase + g * 16, 16)]
            for r in range(16):
                w = ew16.at[jnp.full((16,), r, jnp.int32)].get(
                    mode="promise_in_bounds")
                row = rbase + g * 16 + r
                for j in range(D // 16):
                    sl = pl.ds(j * 16, 16)
                    rowsv[row, sl] = rowsv[row, sl] * w
            return carry2

        lax.fori_loop(0, CHUNK // 16, grp_body, 0)

        for b in range(4):
            @pl.when(par == b)
            def _():
                _scatter_start(c, b)
                nb = (b + 2) % 4

                @pl.when(c + 2 < NCHUNK)
                def _():
                    @pl.when(c >= 2)
                    def _():
                        _scatter_wait(c - 2, nb)

                    _src_wait(c + 2, nb)
                    _gather_start(c + 2, nb)
                    _dstew_start(c + 2, nb)

                    @pl.when(c + 4 < NCHUNK)
                    def _():
                        _src_start(c + 4, b)
        return carry

    lax.fori_loop(0, NCHUNK, chunk_body, 0)

    for k in range(4):
        c = NCHUNK - 4 + k
        _scatter_wait(c, c % 4)

    plsc.subcore_barrier()

    pltpu.sync_copy(agg_s.at[pl.ds(sid * TROWS, TROWS)],
                    agg_out.at[cid, pl.ds(sid * TROWS, TROWS)])

    @pl.when(sid == NS - 1)
    def _():
        pltpu.sync_copy(agg_s.at[pl.ds(NS * TROWS, ZROWS)],
                        agg_out.at[cid, pl.ds(NS * TROWS, ZROWS)])


def _dis_from_partials(pt_blk):
    deg = 1.0 + jnp.sum(pt_blk, axis=1, keepdims=True)
    return lax.rsqrt(deg)


def _hprime_body(x_ref, w_ref, pt_ref, o_ref):
    dis = _dis_from_partials(pt_ref[...])
    h = jnp.dot(x_ref[...], w_ref[...], preferred_element_type=jnp.float32)
    o_ref[...] = h * dis


def _mid_body(agg_ref, hp_ref, pt_ref, b_ref, w_ref, o_ref):
    dis = _dis_from_partials(pt_ref[...])
    tot = agg_ref[0] + agg_ref[1] + hp_ref[...]
    h1 = jnp.maximum(tot * dis + b_ref[...], 0.0)
    o_ref[...] = jnp.dot(h1, w_ref[...], preferred_element_type=jnp.float32) * dis


def _final_body(agg_ref, hp_ref, pt_ref, b_ref, o_ref):
    dis = _dis_from_partials(pt_ref[...])
    tot = agg_ref[0] + agg_ref[1] + hp_ref[...]
    o_ref[...] = tot * dis + b_ref[...]


_row_spec = pl.BlockSpec((BN, D), lambda i: (i, 0))
_pt_spec = pl.BlockSpec((BN, NW), lambda i: (i, 0))
_w_spec = pl.BlockSpec((D, D), lambda i: (0, 0))
_b_spec = pl.BlockSpec((1, D), lambda i: (0, 0))
_agg_spec = pl.BlockSpec((NC, BN, D), lambda i: (0, i, 0))
_out_struct = jax.ShapeDtypeStruct((N, D), jnp.float32)
_GRID = (N // BN,)

_hprime_call = pl.pallas_call(
    _hprime_body, grid=_GRID,
    in_specs=[_row_spec, _w_spec, _pt_spec],
    out_specs=_row_spec, out_shape=_out_struct)

_mid_call = pl.pallas_call(
    _mid_body, grid=_GRID,
    in_specs=[_agg_spec, _row_spec, _pt_spec, _b_spec, _w_spec],
    out_specs=_row_spec, out_shape=_out_struct)

_final_call = pl.pallas_call(
    _final_body, grid=_GRID,
    in_specs=[_agg_spec, _row_spec, _pt_spec, _b_spec],
    out_specs=_row_spec, out_shape=_out_struct)


def kernel(x, edge_index, edge_weight, W1, b1, W2, b2):
    src = edge_index[0]
    dst = edge_index[1]
    b1r = b1.reshape(1, D)
    b2r = b2.reshape(1, D)

    zeros = jnp.zeros((N, D), jnp.float32)

    deg_partials = _deg_kernel(dst, edge_weight)
    pt = deg_partials.T

    h1p = _hprime_call(x, W1, pt)
    agg1 = _agg_kernel(h1p, src, dst, edge_weight, zeros)
    h2p = _mid_call(agg1, h1p, pt, b1r, W2)
    agg2 = _agg_kernel(h2p, src, dst, edge_weight, zeros)
    return _final_call(agg2, h2p, pt, b2r)


# speedup vs baseline: 30.6559x; 1.0095x over previous
import functools

import jax
import jax.numpy as jnp
from jax import lax
from jax.experimental import pallas as pl
from jax.experimental.pallas import tpu as pltpu
from jax.experimental.pallas import tpu_sc as plsc

N = 10000
E = 320000
D = 128

NC = 2
NS = 16
NW = NC * NS
EPW = E // NW
CHUNK = 80
NCHUNK = EPW // CHUNK
TROWS = 624
ZROWS = 16
BN = 1000


def _mesh():
    return plsc.VectorSubcoreMesh(core_axis_name="c", subcore_axis_name="s")


@functools.partial(
    pl.kernel,
    mesh=_mesh(),
    out_type=jax.ShapeDtypeStruct((NW, N), jnp.float32),
    compiler_params=pltpu.CompilerParams(needs_layout_passes=False),
    scratch_types=[
        pltpu.VMEM((EPW,), jnp.int32),
        pltpu.VMEM((EPW,), jnp.float32),
        pltpu.VMEM((N,), jnp.float32),
    ],
)
def _deg_kernel(dst_hbm, ew_hbm, deg_out, dstv, eww, degv):
    cid = lax.axis_index("c")
    sid = lax.axis_index("s")
    wid = cid * NS + sid
    base = wid * EPW

    z16 = jnp.zeros((16,), jnp.float32)

    def zero_body(i, carry):
        degv[pl.ds(i * 16, 16)] = z16
        return carry

    lax.fori_loop(0, N // 16, zero_body, 0)

    pltpu.sync_copy(dst_hbm.at[pl.ds(base, EPW)], dstv)
    pltpu.sync_copy(ew_hbm.at[pl.ds(base, EPW)], eww)

    def grp_body(g, carry):
        d16 = dstv[pl.ds(g * 16, 16)]
        w16 = eww[pl.ds(g * 16, 16)]
        plsc.addupdate_scatter(degv, [d16], w16)
        return carry

    lax.fori_loop(0, EPW // 16, grp_body, 0)

    pltpu.sync_copy(degv, deg_out.at[wid])


@functools.partial(
    pl.kernel,
    mesh=_mesh(),
    out_type=jax.ShapeDtypeStruct((NC, N, D), jnp.float32),
    compiler_params=pltpu.CompilerParams(needs_layout_passes=False),
    scratch_types=[
        pltpu.VMEM((EPW,), jnp.int32),
        pltpu.VMEM((3 * CHUNK, D), jnp.float32),
        pltpu.VMEM((3, CHUNK), jnp.int32),
        pltpu.VMEM((3 * CHUNK,), jnp.float32),
        pltpu.VMEM_SHARED((N, D), jnp.float32),
        pltpu.SemaphoreType.DMA,
        pltpu.SemaphoreType.DMA,
        pltpu.SemaphoreType.DMA,
        pltpu.SemaphoreType.DMA,
        pltpu.SemaphoreType.DMA,
        pltpu.SemaphoreType.DMA,
        pltpu.SemaphoreType.DMA,
        pltpu.SemaphoreType.DMA,
        pltpu.SemaphoreType.DMA,
        pltpu.SemaphoreType.DMA,
    ],
)
def _agg_kernel(h_hbm, src_hbm, dst_hbm, ew_hbm, zeros_hbm, agg_out,
                srcv, rowsv, dstv, eww, agg_s,
                g0, g1, g2, s0, s1, s2, e0, e1, e2, zsem):
    cid = lax.axis_index("c")
    sid = lax.axis_index("s")
    wid = cid * NS + sid
    base = wid * EPW
    gsem = (g0, g1, g2)
    ssem = (s0, s1, s2)
    esem = (e0, e1, e2)

    pltpu.async_copy(zeros_hbm.at[pl.ds(sid * TROWS, TROWS)],
                     agg_s.at[pl.ds(sid * TROWS, TROWS)], zsem)

    @pl.when(sid == NS - 1)
    def _():
        pltpu.async_copy(zeros_hbm.at[pl.ds(NS * TROWS, ZROWS)],
                         agg_s.at[pl.ds(NS * TROWS, ZROWS)], zsem)

    pltpu.sync_copy(src_hbm.at[pl.ds(base, EPW)], srcv)

    def _rows(b):
        return rowsv.at[pl.ds(b * CHUNK, CHUNK)]

    def _gather_start(c, b):
        pltpu.async_copy(h_hbm.at[srcv.at[pl.ds(c * CHUNK, CHUNK)]],
                         _rows(b), gsem[b])

    def _gather_wait(c, b):
        pltpu.make_async_copy(h_hbm.at[srcv.at[pl.ds(c * CHUNK, CHUNK)]],
                              _rows(b), gsem[b]).wait()

    def _dstew_start(c, b):
        pltpu.async_copy(dst_hbm.at[pl.ds(base + c * CHUNK, CHUNK)],
                         dstv.at[b], esem[b])
        pltpu.async_copy(ew_hbm.at[pl.ds(base + c * CHUNK, CHUNK)],
                         eww.at[pl.ds(b * CHUNK, CHUNK)], esem[b])

    def _dstew_wait(c, b):
        pltpu.make_async_copy(dst_hbm.at[pl.ds(base + c * CHUNK, CHUNK)],
                              dstv.at[b], esem[b]).wait()
        pltpu.make_async_copy(ew_hbm.at[pl.ds(base + c * CHUNK, CHUNK)],
                              eww.at[pl.ds(b * CHUNK, CHUNK)], esem[b]).wait()

    def _scatter_start(c, b):
        pltpu.async_copy(_rows(b), agg_s.at[dstv.at[b]], ssem[b], add=True)

    def _scatter_wait(c, b):
        pltpu.make_async_copy(_rows(b), agg_s.at[dstv.at[b]], ssem[b]).wait()

    _gather_start(0, 0)
    _dstew_start(0, 0)
    _gather_start(1, 1)
    _dstew_start(1, 1)

    pltpu.make_async_copy(zeros_hbm.at[pl.ds(sid * TROWS, TROWS)],
                          agg_s.at[pl.ds(sid * TROWS, TROWS)], zsem).wait()

    @pl.when(sid == NS - 1)
    def _():
        pltpu.make_async_copy(zeros_hbm.at[pl.ds(NS * TROWS, ZROWS)],
                              agg_s.at[pl.ds(NS * TROWS, ZROWS)], zsem).wait()

    plsc.subcore_barrier()

    def chunk_body(c, carry):
        par = lax.rem(c, 3)
        for b in range(3):
            @pl.when(par == b)
            def _():
                _gather_wait(c, b)
                _dstew_wait(c, b)

        rbase = par * CHUNK

        def grp_body(g, carry2):
            ew16 = eww[pl.ds(rbase + g * 16, 16)]
            for r in range(16):
                w = ew16.at[jnp.full((16,), r, jnp.int32)].get(
                    mode="promise_in_bounds")
                row = rbase + g * 16 + r
                for j in range(D // 16):
                    sl = pl.ds(j * 16, 16)
                    rowsv[row, sl] = rowsv[row, sl] * w
            return carry2

        lax.fori_loop(0, CHUNK // 16, grp_body, 0)

        for b in range(3):
            @pl.when(par == b)
            def _():
                _scatter_start(c, b)
                nb = (b + 2) % 3

                @pl.when(c + 2 < NCHUNK)
                def _():
                    @pl.when(c >= 1)
                    def _():
                        _scatter_wait(c - 1, nb)

                    _gather_start(c + 2, nb)
                    _dstew_start(c + 2, nb)
        return carry

    lax.fori_loop(0, NCHUNK, chunk_body, 0)

    for k in range(3):
        c = NCHUNK - 3 + k
        _scatter_wait(c, c % 3)

    plsc.subcore_barrier()

    pltpu.sync_copy(agg_s.at[pl.ds(sid * TROWS, TROWS)],
                    agg_out.at[cid, pl.ds(sid * TROWS, TROWS)])

    @pl.when(sid == NS - 1)
    def _():
        pltpu.sync_copy(agg_s.at[pl.ds(NS * TROWS, ZROWS)],
                        agg_out.at[cid, pl.ds(NS * TROWS, ZROWS)])


def _dis_from_partials(pt_blk):
    deg = 1.0 + jnp.sum(pt_blk, axis=1, keepdims=True)
    return lax.rsqrt(deg)


def _hprime_body(x_ref, w_ref, pt_ref, o_ref):
    dis = _dis_from_partials(pt_ref[...])
    h = jnp.dot(x_ref[...], w_ref[...], preferred_element_type=jnp.float32)
    o_ref[...] = h * dis


def _mid_body(agg_ref, hp_ref, pt_ref, b_ref, w_ref, o_ref):
    dis = _dis_from_partials(pt_ref[...])
    tot = agg_ref[0] + agg_ref[1] + hp_ref[...]
    h1 = jnp.maximum(tot * dis + b_ref[...], 0.0)
    o_ref[...] = jnp.dot(h1, w_ref[...], preferred_element_type=jnp.float32) * dis


def _final_body(agg_ref, hp_ref, pt_ref, b_ref, o_ref):
    dis = _dis_from_partials(pt_ref[...])
    tot = agg_ref[0] + agg_ref[1] + hp_ref[...]
    o_ref[...] = tot * dis + b_ref[...]


_row_spec = pl.BlockSpec((BN, D), lambda i: (i, 0))
_pt_spec = pl.BlockSpec((BN, NW), lambda i: (i, 0))
_w_spec = pl.BlockSpec((D, D), lambda i: (0, 0))
_b_spec = pl.BlockSpec((1, D), lambda i: (0, 0))
_agg_spec = pl.BlockSpec((NC, BN, D), lambda i: (0, i, 0))
_out_struct = jax.ShapeDtypeStruct((N, D), jnp.float32)
_GRID = (N // BN,)

_hprime_call = pl.pallas_call(
    _hprime_body, grid=_GRID,
    in_specs=[_row_spec, _w_spec, _pt_spec],
    out_specs=_row_spec, out_shape=_out_struct)

_mid_call = pl.pallas_call(
    _mid_body, grid=_GRID,
    in_specs=[_agg_spec, _row_spec, _pt_spec, _b_spec, _w_spec],
    out_specs=_row_spec, out_shape=_out_struct)

_final_call = pl.pallas_call(
    _final_body, grid=_GRID,
    in_specs=[_agg_spec, _row_spec, _pt_spec, _b_spec],
    out_specs=_row_spec, out_shape=_out_struct)


def kernel(x, edge_index, edge_weight, W1, b1, W2, b2):
    src = edge_index[0]
    dst = edge_index[1]
    b1r = b1.reshape(1, D)
    b2r = b2.reshape(1, D)

    zeros = jnp.zeros((N, D), jnp.float32)

    deg_partials = _deg_kernel(dst, edge_weight)
    pt = deg_partials.T

    h1p = _hprime_call(x, W1, pt)
    agg1 = _agg_kernel(h1p, src, dst, edge_weight, zeros)
    h2p = _mid_call(agg1, h1p, pt, b1r, W2)
    agg2 = _agg_kernel(h2p, src, dst, edge_weight, zeros)
    return _final_call(agg2, h2p, pt, b2r)
